# baseline (speedup 1.0000x reference)
import jax
import jax.numpy as jnp
from jax.experimental import pallas as pl
from jax.experimental.pallas import tpu as pltpu

NSEG = 64


def _pick_tile(E):
    for t in (10000, 8000, 4000, 3200, 2560, 2000, 1600, 1280, 1000, 800, 640, 500, 320, 200, 160, 8):
        if E % t == 0:
            return t
    return E


def _fused_body(seg_ref, src_ref, dest_ref, ea_ref, u_ref,
                W0a, W0b, W0c, W0d, b0, W1, b1, W2, b2,
                V0a, V0b, c0, V1, c1,
                out_ref, wts_ref, m_out, s_out,
                m_scr, s_scr):
    i = pl.program_id(0)
    n = pl.num_programs(0)
    f32 = jnp.float32
    bf16 = jnp.bfloat16
    tile = seg_ref.shape[-1]

    @pl.when(i == 0)
    def _init():
        m_scr[...] = jnp.full(m_scr.shape, -jnp.inf, f32)
        s_scr[...] = jnp.zeros(s_scr.shape, f32)

    seg = seg_ref[0]
    mask = jax.lax.broadcasted_iota(jnp.int32, (NSEG, tile), 0) == seg
    maskb = mask.astype(bf16)

    uW = jnp.dot(u_ref[...], W0d[...], preferred_element_type=f32)
    uV = jnp.dot(u_ref[...], V0b[...], preferred_element_type=f32)
    dn_seg = (((0,), (0,)), ((), ()))
    ue_W = jax.lax.dot_general(maskb, uW.astype(bf16), dn_seg,
                               preferred_element_type=f32)
    ue_V = jax.lax.dot_general(maskb, uV.astype(bf16), dn_seg,
                               preferred_element_type=f32)

    h = jnp.dot(src_ref[...].astype(bf16), W0a[...].astype(bf16),
                preferred_element_type=f32)
    h = h + jnp.dot(dest_ref[...].astype(bf16), W0b[...].astype(bf16),
                    preferred_element_type=f32)
    h = h + jnp.dot(ea_ref[...].astype(bf16), W0c[...].astype(bf16),
                    preferred_element_type=f32)
    h = jax.nn.relu(h + ue_W + b0[...])
    h = jax.nn.relu(jnp.dot(h.astype(bf16), W1[...].astype(bf16),
                            preferred_element_type=f32) + b1[...])
    out_ref[...] = jnp.dot(h.astype(bf16), W2[...].astype(bf16),
                           preferred_element_type=f32) + b2[...]

    wh = jax.nn.relu(jnp.dot(ea_ref[...].astype(bf16), V0a[...].astype(bf16),
                             preferred_element_type=f32) + ue_V + c0[...])
    dn_row = (((0,), (1,)), ((), ()))
    wts_row = jax.lax.dot_general(V1[...].astype(bf16), wh.astype(bf16), dn_row,
                                  preferred_element_type=f32) + c1[0, 0]
    wts_ref[0] = wts_row

    masked = jnp.where(mask, wts_row, -jnp.inf)
    m_tile = jnp.max(masked, axis=1, keepdims=True)
    m_old = m_scr[...]
    m_new = jnp.maximum(m_old, m_tile)
    scale = jnp.where(m_old == -jnp.inf, 0.0, jnp.exp(m_old - m_new))
    ex = jnp.where(mask, jnp.exp(wts_row - m_new), 0.0)
    s_scr[...] = s_scr[...] * scale + jnp.sum(ex, axis=1, keepdims=True)
    m_scr[...] = m_new

    @pl.when(i == n - 1)
    def _fin():
        m_fin = m_scr[...]
        s_fin = s_scr[...]
        empty = m_fin == -jnp.inf
        m_out[...] = jnp.where(empty, 0.0, m_fin)
        s_out[...] = jnp.where(empty, 1.0, s_fin)


def _norm_body(seg_ref, wts_ref, m_ref, s_ref, out_ref):
    tile = seg_ref.shape[-1]
    seg = seg_ref[0]
    mask = jax.lax.broadcasted_iota(jnp.int32, (NSEG, tile), 0) == seg
    m = jnp.sum(jnp.where(mask, m_ref[...], 0.0), axis=0, keepdims=True)
    s = jnp.sum(jnp.where(mask, s_ref[...], 0.0), axis=0, keepdims=True)
    out_ref[0] = jnp.exp(wts_ref[0] - m) / s


def kernel(src, dest, edge_attr, u, edge_batch, W0, b0, W1, b1, W2, b2, V0, c0, V1, c1):
    E, node_dim = src.shape
    edge_dim = edge_attr.shape[1]
    global_dim = u.shape[1]
    hidden = W1.shape[0]
    out_dim = W2.shape[1]
    f32 = jnp.float32

    T = _pick_tile(E)
    nb = E // T
    seg3 = edge_batch.astype(jnp.int32).reshape(nb, 1, T)

    W0a = W0[:node_dim]
    W0b = W0[node_dim:2 * node_dim]
    W0c = W0[2 * node_dim:2 * node_dim + edge_dim]
    W0d = W0[2 * node_dim + edge_dim:]
    V0a = V0[:edge_dim]
    V0b = V0[edge_dim:]
    b0r = b0.reshape(1, hidden)
    b1r = b1.reshape(1, hidden)
    b2r = b2.reshape(1, out_dim)
    c0r = c0.reshape(1, hidden)
    c1r = c1.reshape(1, 1)

    full = lambda shape: pl.BlockSpec(shape, lambda i: (0,) * len(shape))
    row_spec = pl.BlockSpec((1, 1, T), lambda i: (i, 0, 0))

    out, wts_rows, m, s = pl.pallas_call(
        _fused_body,
        grid=(nb,),
        in_specs=[
            row_spec,
            pl.BlockSpec((T, node_dim), lambda i: (i, 0)),
            pl.BlockSpec((T, node_dim), lambda i: (i, 0)),
            pl.BlockSpec((T, edge_dim), lambda i: (i, 0)),
            full((NSEG, global_dim)),
            full((node_dim, hidden)),
            full((node_dim, hidden)),
            full((edge_dim, hidden)),
            full((global_dim, hidden)),
            full((1, hidden)),
            full((hidden, hidden)),
            full((1, hidden)),
            full((hidden, out_dim)),
            full((1, out_dim)),
            full((edge_dim, hidden)),
            full((global_dim, hidden)),
            full((1, hidden)),
            full((hidden, 1)),
            full((1, 1)),
        ],
        out_specs=[
            pl.BlockSpec((T, out_dim), lambda i: (i, 0)),
            row_spec,
            full((NSEG, 1)),
            full((NSEG, 1)),
        ],
        out_shape=[
            jax.ShapeDtypeStruct((E, out_dim), f32),
            jax.ShapeDtypeStruct((nb, 1, T), f32),
            jax.ShapeDtypeStruct((NSEG, 1), f32),
            jax.ShapeDtypeStruct((NSEG, 1), f32),
        ],
        scratch_shapes=[
            pltpu.VMEM((NSEG, 1), f32),
            pltpu.VMEM((NSEG, 1), f32),
        ],
        compiler_params=pltpu.CompilerParams(
            dimension_semantics=("arbitrary",)),
    )(seg3, src, dest, edge_attr, u, W0a, W0b, W0c, W0d, b0r,
      W1, b1r, W2, b2r, V0a, V0b, c0r, V1, c1r)

    norm_rows = pl.pallas_call(
        _norm_body,
        grid=(nb,),
        in_specs=[row_spec, row_spec, full((NSEG, 1)), full((NSEG, 1))],
        out_specs=row_spec,
        out_shape=jax.ShapeDtypeStruct((nb, 1, T), f32),
        compiler_params=pltpu.CompilerParams(
            dimension_semantics=("arbitrary",)),
    )(seg3, wts_rows, m, s)

    return (out, norm_rows.reshape(E, 1), wts_rows.reshape(E, 1))


# speedup vs baseline: 1.1600x; 1.1600x over previous
import jax
import jax.numpy as jnp
from jax.experimental import pallas as pl
from jax.experimental.pallas import tpu as pltpu

NSEG = 64


def _pick_tile(E):
    for t in (8000, 4000, 3200, 2560, 2000, 1600, 1280, 1000, 800, 640, 500, 320, 200, 160, 8):
        if E % t == 0:
            return t
    return E


def _fused_body(seg_ref, src_ref, dest_ref, ea_ref, u_ref,
                W0a, W0b, W0c, W0d, b0, W1, b1, W2, b2,
                V0a, V0b, c0, V1, c1,
                out_ref, wts_ref, m_out, s_out,
                m_scr, s_scr):
    i = pl.program_id(0)
    n = pl.num_programs(0)
    f32 = jnp.float32
    bf16 = jnp.bfloat16
    tile = seg_ref.shape[-1]

    @pl.when(i == 0)
    def _init():
        m_scr[...] = jnp.full(m_scr.shape, -jnp.inf, f32)
        s_scr[...] = jnp.zeros(s_scr.shape, f32)

    seg = seg_ref[0]
    mask = jax.lax.broadcasted_iota(jnp.int32, (NSEG, tile), 0) == seg
    maskb = mask.astype(bf16)

    uW = jnp.dot(u_ref[...], W0d[...], preferred_element_type=f32)
    uV = jnp.dot(u_ref[...], V0b[...], preferred_element_type=f32)
    dn_seg = (((0,), (0,)), ((), ()))
    ue_W = jax.lax.dot_general(maskb, uW.astype(bf16), dn_seg,
                               preferred_element_type=f32)
    ue_V = jax.lax.dot_general(maskb, uV.astype(bf16), dn_seg,
                               preferred_element_type=f32)

    h = jnp.dot(src_ref[...].astype(bf16), W0a[...].astype(bf16),
                preferred_element_type=f32)
    h = h + jnp.dot(dest_ref[...].astype(bf16), W0b[...].astype(bf16),
                    preferred_element_type=f32)
    h = h + jnp.dot(ea_ref[...].astype(bf16), W0c[...].astype(bf16),
                    preferred_element_type=f32)
    h = jax.nn.relu(h + ue_W + b0[...])
    h = jax.nn.relu(jnp.dot(h.astype(bf16), W1[...].astype(bf16),
                            preferred_element_type=f32) + b1[...])
    out_ref[...] = jnp.dot(h.astype(bf16), W2[...].astype(bf16),
                           preferred_element_type=f32) + b2[...]

    wh = jax.nn.relu(jnp.dot(ea_ref[...].astype(bf16), V0a[...].astype(bf16),
                             preferred_element_type=f32) + ue_V + c0[...])
    dn_row = (((0,), (1,)), ((), ()))
    wts_row = jax.lax.dot_general(V1[...].astype(bf16), wh.astype(bf16), dn_row,
                                  preferred_element_type=f32) + c1[0, 0]
    wts_ref[0] = wts_row

    m_old = m_scr[0, 0]
    m_new = jnp.maximum(m_old, jnp.max(wts_row))
    ex_row = jnp.exp(wts_row - m_new)
    dn_stat = (((1,), (1,)), ((), ()))
    s_tile = jax.lax.dot_general(mask.astype(f32), ex_row, dn_stat,
                                 preferred_element_type=f32)
    s_scr[...] = s_scr[...] * jnp.exp(m_old - m_new) + s_tile
    m_scr[...] = jnp.broadcast_to(m_new, m_scr.shape)

    @pl.when(i == n - 1)
    def _fin():
        m_out[...] = jnp.broadcast_to(m_scr[0, 0], m_out.shape)
        s_out[...] = jnp.where(s_scr[...] == 0.0, 1.0, s_scr[...])


def _norm_body(seg_ref, wts_ref, m_ref, s_ref, out_ref):
    tile = seg_ref.shape[-1]
    seg = seg_ref[0]
    mask = jax.lax.broadcasted_iota(jnp.int32, (NSEG, tile), 0) == seg
    m = jnp.sum(jnp.where(mask, m_ref[...], 0.0), axis=0, keepdims=True)
    s = jnp.sum(jnp.where(mask, s_ref[...], 0.0), axis=0, keepdims=True)
    out_ref[0] = jnp.exp(wts_ref[0] - m) / s


def kernel(src, dest, edge_attr, u, edge_batch, W0, b0, W1, b1, W2, b2, V0, c0, V1, c1):
    E, node_dim = src.shape
    edge_dim = edge_attr.shape[1]
    global_dim = u.shape[1]
    hidden = W1.shape[0]
    out_dim = W2.shape[1]
    f32 = jnp.float32

    T = _pick_tile(E)
    nb = E // T
    seg3 = edge_batch.astype(jnp.int32).reshape(nb, 1, T)

    W0a = W0[:node_dim]
    W0b = W0[node_dim:2 * node_dim]
    W0c = W0[2 * node_dim:2 * node_dim + edge_dim]
    W0d = W0[2 * node_dim + edge_dim:]
    V0a = V0[:edge_dim]
    V0b = V0[edge_dim:]
    b0r = b0.reshape(1, hidden)
    b1r = b1.reshape(1, hidden)
    b2r = b2.reshape(1, out_dim)
    c0r = c0.reshape(1, hidden)
    c1r = c1.reshape(1, 1)

    full = lambda shape: pl.BlockSpec(shape, lambda i: (0,) * len(shape))
    row_spec = pl.BlockSpec((1, 1, T), lambda i: (i, 0, 0))

    out, wts_rows, m, s = pl.pallas_call(
        _fused_body,
        grid=(nb,),
        in_specs=[
            row_spec,
            pl.BlockSpec((T, node_dim), lambda i: (i, 0)),
            pl.BlockSpec((T, node_dim), lambda i: (i, 0)),
            pl.BlockSpec((T, edge_dim), lambda i: (i, 0)),
            full((NSEG, global_dim)),
            full((node_dim, hidden)),
            full((node_dim, hidden)),
            full((edge_dim, hidden)),
            full((global_dim, hidden)),
            full((1, hidden)),
            full((hidden, hidden)),
            full((1, hidden)),
            full((hidden, out_dim)),
            full((1, out_dim)),
            full((edge_dim, hidden)),
            full((global_dim, hidden)),
            full((1, hidden)),
            full((hidden, 1)),
            full((1, 1)),
        ],
        out_specs=[
            pl.BlockSpec((T, out_dim), lambda i: (i, 0)),
            row_spec,
            full((NSEG, 1)),
            full((NSEG, 1)),
        ],
        out_shape=[
            jax.ShapeDtypeStruct((E, out_dim), f32),
            jax.ShapeDtypeStruct((nb, 1, T), f32),
            jax.ShapeDtypeStruct((NSEG, 1), f32),
            jax.ShapeDtypeStruct((NSEG, 1), f32),
        ],
        scratch_shapes=[
            pltpu.VMEM((NSEG, 1), f32),
            pltpu.VMEM((NSEG, 1), f32),
        ],
        compiler_params=pltpu.CompilerParams(
            dimension_semantics=("arbitrary",)),
    )(seg3, src, dest, edge_attr, u, W0a, W0b, W0c, W0d, b0r,
      W1, b1r, W2, b2r, V0a, V0b, c0r, V1, c1r)

    norm_rows = pl.pallas_call(
        _norm_body,
        grid=(nb,),
        in_specs=[row_spec, row_spec, full((NSEG, 1)), full((NSEG, 1))],
        out_specs=row_spec,
        out_shape=jax.ShapeDtypeStruct((nb, 1, T), f32),
        compiler_params=pltpu.CompilerParams(
            dimension_semantics=("arbitrary",)),
    )(seg3, wts_rows, m, s)

    return (out, norm_rows.reshape(E, 1), wts_rows.reshape(E, 1))


# speedup vs baseline: 1.3567x; 1.1696x over previous
import jax
import jax.numpy as jnp
from jax.experimental import pallas as pl
from jax.experimental.pallas import tpu as pltpu

NSEG = 64


def _pick_tile(E):
    for t in (8000, 4000, 3200, 2560, 2000, 1600, 1280, 1000, 800, 640, 500, 320, 200, 160, 8):
        if E % t == 0:
            return t
    return E


def _fused_body(seg_ref, src_ref, dest_ref, ea_ref, u_ref,
                W0a, W0b, W0c, W0d, b0, W1, b1, W2, b2,
                V0a, V0b, c0, V1, c1,
                out_ref, wts_ref, m_out, s_out,
                m_scr, s_scr, xcat):
    i = pl.program_id(0)
    n = pl.num_programs(0)
    f32 = jnp.float32
    bf16 = jnp.bfloat16
    tile = seg_ref.shape[-1]

    @pl.when(i == 0)
    def _init():
        m_scr[...] = jnp.full(m_scr.shape, -jnp.inf, f32)
        s_scr[...] = jnp.zeros(s_scr.shape, f32)

    seg = seg_ref[0]
    mask = jax.lax.broadcasted_iota(jnp.int32, (NSEG, tile), 0) == seg
    maskb = mask.astype(bf16)

    nd = src_ref.shape[1]
    ed = ea_ref.shape[1]
    eye = (jax.lax.broadcasted_iota(jnp.int32, (NSEG, NSEG), 0) ==
           jax.lax.broadcasted_iota(jnp.int32, (NSEG, NSEG), 1)).astype(bf16)
    dn_t = (((0,), (0,)), ((), ()))
    oh_col = jax.lax.dot_general(maskb, eye, dn_t,
                                 preferred_element_type=f32).astype(bf16)
    xcat[:, 0:nd] = src_ref[...].astype(bf16)
    xcat[:, nd:2 * nd] = dest_ref[...].astype(bf16)
    xcat[:, 2 * nd:2 * nd + ed] = ea_ref[...].astype(bf16)
    xcat[:, 2 * nd + ed:] = oh_col

    uW = jnp.dot(u_ref[...], W0d[...], preferred_element_type=f32)
    uV = jnp.dot(u_ref[...], V0b[...], preferred_element_type=f32)
    Wcat = jnp.concatenate([W0a[...].astype(bf16), W0b[...].astype(bf16),
                            W0c[...].astype(bf16), uW.astype(bf16)], axis=0)
    Vcat = jnp.concatenate([V0a[...].astype(bf16), uV.astype(bf16)], axis=0)

    h = jax.nn.relu(jnp.dot(xcat[...], Wcat, preferred_element_type=f32) + b0[...])
    h = jax.nn.relu(jnp.dot(h.astype(bf16), W1[...].astype(bf16),
                            preferred_element_type=f32) + b1[...])
    out_ref[...] = jnp.dot(h.astype(bf16), W2[...].astype(bf16),
                           preferred_element_type=f32) + b2[...]

    wh = jax.nn.relu(jnp.dot(xcat[:, 2 * nd:], Vcat,
                             preferred_element_type=f32) + c0[...])
    dn_row = (((0,), (1,)), ((), ()))
    wts_row = jax.lax.dot_general(V1[...].astype(bf16), wh.astype(bf16), dn_row,
                                  preferred_element_type=f32) + c1[0, 0]
    wts_ref[0] = wts_row

    m_old = m_scr[0, 0]
    m_new = jnp.maximum(m_old, jnp.max(wts_row))
    ex_row = jnp.exp(wts_row - m_new)
    dn_stat = (((1,), (1,)), ((), ()))
    s_tile = jax.lax.dot_general(mask.astype(f32), ex_row, dn_stat,
                                 preferred_element_type=f32)
    s_scr[...] = s_scr[...] * jnp.exp(m_old - m_new) + s_tile
    m_scr[...] = jnp.broadcast_to(m_new, m_scr.shape)

    @pl.when(i == n - 1)
    def _fin():
        m_out[...] = jnp.broadcast_to(m_scr[0, 0], m_out.shape)
        s_out[...] = jnp.where(s_scr[...] == 0.0, 1.0, s_scr[...])


def _norm_body(seg_ref, wts_ref, m_ref, s_ref, out_ref):
    tile = seg_ref.shape[-1]
    seg = seg_ref[0]
    mask = jax.lax.broadcasted_iota(jnp.int32, (NSEG, tile), 0) == seg
    m = jnp.sum(jnp.where(mask, m_ref[...], 0.0), axis=0, keepdims=True)
    s = jnp.sum(jnp.where(mask, s_ref[...], 0.0), axis=0, keepdims=True)
    out_ref[0] = jnp.exp(wts_ref[0] - m) / s


def kernel(src, dest, edge_attr, u, edge_batch, W0, b0, W1, b1, W2, b2, V0, c0, V1, c1):
    E, node_dim = src.shape
    edge_dim = edge_attr.shape[1]
    global_dim = u.shape[1]
    hidden = W1.shape[0]
    out_dim = W2.shape[1]
    f32 = jnp.float32

    T = _pick_tile(E)
    nb = E // T
    seg3 = edge_batch.astype(jnp.int32).reshape(nb, 1, T)

    W0a = W0[:node_dim]
    W0b = W0[node_dim:2 * node_dim]
    W0c = W0[2 * node_dim:2 * node_dim + edge_dim]
    W0d = W0[2 * node_dim + edge_dim:]
    V0a = V0[:edge_dim]
    V0b = V0[edge_dim:]
    b0r = b0.reshape(1, hidden)
    b1r = b1.reshape(1, hidden)
    b2r = b2.reshape(1, out_dim)
    c0r = c0.reshape(1, hidden)
    c1r = c1.reshape(1, 1)

    full = lambda shape: pl.BlockSpec(shape, lambda i: (0,) * len(shape))
    row_spec = pl.BlockSpec((1, 1, T), lambda i: (i, 0, 0))

    out, wts_rows, m, s = pl.pallas_call(
        _fused_body,
        grid=(nb,),
        in_specs=[
            row_spec,
            pl.BlockSpec((T, node_dim), lambda i: (i, 0)),
            pl.BlockSpec((T, node_dim), lambda i: (i, 0)),
            pl.BlockSpec((T, edge_dim), lambda i: (i, 0)),
            full((NSEG, global_dim)),
            full((node_dim, hidden)),
            full((node_dim, hidden)),
            full((edge_dim, hidden)),
            full((global_dim, hidden)),
            full((1, hidden)),
            full((hidden, hidden)),
            full((1, hidden)),
            full((hidden, out_dim)),
            full((1, out_dim)),
            full((edge_dim, hidden)),
            full((global_dim, hidden)),
            full((1, hidden)),
            full((hidden, 1)),
            full((1, 1)),
        ],
        out_specs=[
            pl.BlockSpec((T, out_dim), lambda i: (i, 0)),
            row_spec,
            full((NSEG, 1)),
            full((NSEG, 1)),
        ],
        out_shape=[
            jax.ShapeDtypeStruct((E, out_dim), f32),
            jax.ShapeDtypeStruct((nb, 1, T), f32),
            jax.ShapeDtypeStruct((NSEG, 1), f32),
            jax.ShapeDtypeStruct((NSEG, 1), f32),
        ],
        scratch_shapes=[
            pltpu.VMEM((NSEG, 1), f32),
            pltpu.VMEM((NSEG, 1), f32),
            pltpu.VMEM((T, 2 * node_dim + edge_dim + NSEG), jnp.bfloat16),
        ],
        compiler_params=pltpu.CompilerParams(
            dimension_semantics=("arbitrary",)),
    )(seg3, src, dest, edge_attr, u, W0a, W0b, W0c, W0d, b0r,
      W1, b1r, W2, b2r, V0a, V0b, c0r, V1, c1r)

    norm_rows = pl.pallas_call(
        _norm_body,
        grid=(nb,),
        in_specs=[row_spec, row_spec, full((NSEG, 1)), full((NSEG, 1))],
        out_specs=row_spec,
        out_shape=jax.ShapeDtypeStruct((nb, 1, T), f32),
        compiler_params=pltpu.CompilerParams(
            dimension_semantics=("arbitrary",)),
    )(seg3, wts_rows, m, s)

    return (out, norm_rows.reshape(E, 1), wts_rows.reshape(E, 1))
